# Initial kernel scaffold; baseline (speedup 1.0000x reference)
import jax
import jax.numpy as jnp
from jax.experimental import pallas as pl


def kernel(x, bn_W, bn_b, gc_W, gc_b):
    raise NotImplementedError("write your pallas kernel here")


# speedup vs baseline: 11.4505x; 11.4505x over previous
import functools

import jax
import jax.numpy as jnp
import numpy as np
from jax.experimental import pallas as pl
from jax.experimental.pallas import tpu as pltpu

_E = 1024
_TOP_K = 64
_INT32_MIN = np.int32(-(2 ** 31))
_INT32_MAXP = np.int32(2 ** 31 - 1)


def _bottleneck_kernel(x_ref, w_ref, b_ref, g_ref):
    g_ref[...] = jnp.tanh(
        jnp.dot(x_ref[...], w_ref[...], preferred_element_type=jnp.float32)
        + b_ref[...]
    )


def _attn_kernel(top_k, g_ref, h_ref, gcw_ref, gcb_ref, out_ref):
    g = g_ref[...]
    h = h_ref[...]
    e_dim = g.shape[0]

    s = jax.lax.dot_general(
        g, g, (((1,), (1,)), ((), ())), preferred_element_type=jnp.float32
    )

    m = jnp.max(s, axis=-1, keepdims=True)
    ex = jnp.exp(s - m)
    denom = jnp.sum(ex, axis=-1, keepdims=True)

    bits = jax.lax.bitcast_convert_type(s, jnp.int32)
    skey = jnp.where(bits >= 0, bits, bits ^ _INT32_MAXP)

    def body(i, prefix):
        bitmask = jnp.left_shift(jnp.int32(1), jnp.int32(31) - i)
        trial = prefix | bitmask
        strial = trial ^ _INT32_MIN
        cnt = jnp.sum((skey >= strial).astype(jnp.int32), axis=-1,
                      keepdims=True)
        return jnp.where(cnt >= top_k, trial, prefix)

    prefix0 = jnp.zeros((e_dim, 1), jnp.int32)
    prefix = jax.lax.fori_loop(0, 32, body, prefix0)
    sthr = prefix ^ _INT32_MIN

    keep = skey >= sthr
    row_ids = jax.lax.broadcasted_iota(jnp.int32, (e_dim, e_dim), 0)
    col_ids = jax.lax.broadcasted_iota(jnp.int32, (e_dim, e_dim), 1)
    keep = jnp.logical_and(keep, row_ids != col_ids)

    a = jnp.where(keep, ex, 0.0) / denom

    y = jnp.dot(a, h, preferred_element_type=jnp.float32) + h
    z = jnp.dot(y, gcw_ref[...], preferred_element_type=jnp.float32) \
        + gcb_ref[...]
    out_ref[...] = jnp.maximum(z, 0.0)


@jax.jit
def kernel(x, bn_W, bn_b, gc_W, gc_b):
    n, in_f = x.shape
    b = n // _E
    bn_f = bn_W.shape[1]
    out_f = gc_W.shape[1]

    row_blk = min(4096, n)
    n_blks = n // row_blk
    g = pl.pallas_call(
        _bottleneck_kernel,
        grid=(n_blks,),
        in_specs=[
            pl.BlockSpec((row_blk, in_f), lambda i: (i, 0)),
            pl.BlockSpec((in_f, bn_f), lambda i: (0, 0)),
            pl.BlockSpec((1, bn_f), lambda i: (0, 0)),
        ],
        out_specs=pl.BlockSpec((row_blk, bn_f), lambda i: (i, 0)),
        out_shape=jax.ShapeDtypeStruct((n, bn_f), jnp.float32),
    )(x, bn_W, bn_b.reshape(1, bn_f))

    out = pl.pallas_call(
        functools.partial(_attn_kernel, _TOP_K),
        grid=(b,),
        in_specs=[
            pl.BlockSpec((_E, bn_f), lambda i: (i, 0)),
            pl.BlockSpec((_E, in_f), lambda i: (i, 0)),
            pl.BlockSpec((in_f, out_f), lambda i: (0, 0)),
            pl.BlockSpec((1, out_f), lambda i: (0, 0)),
        ],
        out_specs=pl.BlockSpec((_E, out_f), lambda i: (i, 0)),
        out_shape=jax.ShapeDtypeStruct((n, out_f), jnp.float32),
        compiler_params=pltpu.CompilerParams(
            dimension_semantics=("arbitrary",),
        ),
    )(g, x, gc_W, gc_b.reshape(1, out_f))

    return out.reshape(b, _E, out_f)


# speedup vs baseline: 13.6355x; 1.1908x over previous
import functools

import jax
import jax.numpy as jnp
import numpy as np
from jax.experimental import pallas as pl
from jax.experimental.pallas import tpu as pltpu

_E = 1024
_TOP_K = 64
_INT32_MIN = np.int32(-(2 ** 31))
_INT32_MAXP = np.int32(2 ** 31 - 1)


def _search16(key16, top_k):
    e_dim = key16.shape[0]

    def body(i, prefix):
        bitmask = jnp.left_shift(jnp.int32(1), jnp.int32(15) - i)
        trial = prefix | bitmask
        strial = (trial - 32768).astype(jnp.int16)
        cond = (key16 >= strial).astype(jnp.int16)
        half = cond.shape[1] // 2
        c = cond[:, :half] + cond[:, half:]
        c = c[:, : half // 2] + c[:, half // 2:]
        c = c[:, : half // 4] + c[:, half // 4:]
        cnt = jnp.sum(c.astype(jnp.int32), axis=-1, keepdims=True)
        return jnp.where(cnt >= top_k, trial, prefix)

    prefix0 = jnp.zeros((e_dim, 1), jnp.int32)
    prefix = jax.lax.fori_loop(0, 16, body, prefix0)
    return (prefix - 32768).astype(jnp.int16)


def _attn_kernel(top_k, x_ref, bnw_ref, bnb_ref, gcw_ref, gcb_ref, out_ref):
    h = x_ref[...]
    e_dim = h.shape[0]

    g = jnp.tanh(
        jnp.dot(h, bnw_ref[...], preferred_element_type=jnp.float32)
        + bnb_ref[...]
    )

    s = jax.lax.dot_general(
        g, g, (((1,), (1,)), ((), ())), preferred_element_type=jnp.float32
    )

    m = jnp.max(s, axis=-1, keepdims=True)
    ex = jnp.exp(s - m)
    denom = jnp.sum(ex, axis=-1, keepdims=True)

    bits = jax.lax.bitcast_convert_type(s, jnp.int32)
    ukey = jnp.where(bits >= 0, bits, bits ^ _INT32_MAXP) ^ _INT32_MIN

    hi = (jax.lax.shift_right_logical(ukey, 16) - 32768).astype(jnp.int16)
    lo = ((ukey & 0xFFFF) - 32768).astype(jnp.int16)

    th_hi = _search16(hi, top_k)

    lo_masked = jnp.where(
        hi > th_hi, jnp.int16(32767),
        jnp.where(hi < th_hi, jnp.int16(-32768), lo))
    th_lo = _search16(lo_masked, top_k)

    keep = lo_masked >= th_lo
    row_ids = jax.lax.broadcasted_iota(jnp.int32, (e_dim, e_dim), 0)
    col_ids = jax.lax.broadcasted_iota(jnp.int32, (e_dim, e_dim), 1)
    keep = jnp.logical_and(keep, row_ids != col_ids)

    am = jnp.where(keep, ex, 0.0)

    y = jnp.dot(am, h, preferred_element_type=jnp.float32) / denom + h
    z = jnp.dot(y, gcw_ref[...], preferred_element_type=jnp.float32) \
        + gcb_ref[...]
    out_ref[...] = jnp.maximum(z, 0.0)


@jax.jit
def kernel(x, bn_W, bn_b, gc_W, gc_b):
    n, in_f = x.shape
    b = n // _E
    bn_f = bn_W.shape[1]
    out_f = gc_W.shape[1]

    out = pl.pallas_call(
        functools.partial(_attn_kernel, _TOP_K),
        grid=(b,),
        in_specs=[
            pl.BlockSpec((_E, in_f), lambda i: (i, 0)),
            pl.BlockSpec((in_f, bn_f), lambda i: (0, 0)),
            pl.BlockSpec((1, bn_f), lambda i: (0, 0)),
            pl.BlockSpec((in_f, out_f), lambda i: (0, 0)),
            pl.BlockSpec((1, out_f), lambda i: (0, 0)),
        ],
        out_specs=pl.BlockSpec((_E, out_f), lambda i: (i, 0)),
        out_shape=jax.ShapeDtypeStruct((n, out_f), jnp.float32),
        compiler_params=pltpu.CompilerParams(
            dimension_semantics=("arbitrary",),
        ),
    )(x, bn_W, bn_b.reshape(1, bn_f), gc_W, gc_b.reshape(1, out_f))

    return out.reshape(b, _E, out_f)


# speedup vs baseline: 22.2246x; 1.6299x over previous
import functools

import jax
import jax.numpy as jnp
import numpy as np
from jax.experimental import pallas as pl
from jax.experimental.pallas import tpu as pltpu

_E = 1024
_TOP_K = 64
_INT32_MIN = np.int32(-(2 ** 31))
_INT32_MAXP = np.int32(2 ** 31 - 1)


def _search16_cols(key16, top_k):
    e_dim = key16.shape[0]

    def body(i, prefix):
        bitmask = jnp.left_shift(jnp.int32(1), jnp.int32(15) - i)
        trial = prefix | bitmask
        strial = (trial - 32768).astype(jnp.int16)
        cond = (key16 >= strial).astype(jnp.int16)
        c = cond[: e_dim // 2] + cond[e_dim // 2:]
        c = c[: e_dim // 4] + c[e_dim // 4:]
        c = c[: e_dim // 8] + c[e_dim // 8:]
        c = c[: e_dim // 16] + c[e_dim // 16:]
        cnt = jnp.sum(c.astype(jnp.int32), axis=0, keepdims=True)
        return jnp.where(cnt >= top_k, trial, prefix)

    prefix0 = jnp.zeros((1, key16.shape[1]), jnp.int32)
    prefix = jax.lax.fori_loop(0, 16, body, prefix0)
    return (prefix - 32768).astype(jnp.int16)


def _attn_kernel(top_k, x_ref, bnw_ref, bnb_ref, gcw_ref, gcb_ref, out_ref):
    h = x_ref[...]
    e_dim = h.shape[0]

    g = jnp.tanh(
        jnp.dot(h, bnw_ref[...], preferred_element_type=jnp.float32)
        + bnb_ref[...]
    )

    s = jax.lax.dot_general(
        g, g, (((1,), (1,)), ((), ())), preferred_element_type=jnp.float32
    )

    m = jnp.max(s, axis=-1, keepdims=True)
    ex = jnp.exp(s - m)
    denom = jnp.sum(ex, axis=-1, keepdims=True)

    bits = jax.lax.bitcast_convert_type(s, jnp.int32)
    ukey = jnp.where(bits >= 0, bits, bits ^ _INT32_MAXP) ^ _INT32_MIN

    hi = (jax.lax.shift_right_logical(ukey, 16) - 32768).astype(jnp.int16)
    lo = ((ukey & 0xFFFF) - 32768).astype(jnp.int16)

    th_hi = _search16_cols(hi, top_k)

    lo_m_cols = jnp.where(
        hi > th_hi, jnp.int16(32767),
        jnp.where(hi < th_hi, jnp.int16(-32768), lo))
    th_lo = _search16_cols(lo_m_cols, top_k)

    th_hi_c = jnp.transpose(th_hi.astype(jnp.int32)).astype(jnp.int16)
    th_lo_c = jnp.transpose(th_lo.astype(jnp.int32)).astype(jnp.int16)
    lo_m_rows = jnp.where(
        hi > th_hi_c, jnp.int16(32767),
        jnp.where(hi < th_hi_c, jnp.int16(-32768), lo))
    keep = lo_m_rows >= th_lo_c

    row_ids = jax.lax.broadcasted_iota(jnp.int32, (e_dim, e_dim), 0)
    col_ids = jax.lax.broadcasted_iota(jnp.int32, (e_dim, e_dim), 1)
    keep = jnp.logical_and(keep, row_ids != col_ids)

    am = jnp.where(keep, ex, 0.0)

    y = jnp.dot(am, h, preferred_element_type=jnp.float32) / denom + h
    z = jnp.dot(y, gcw_ref[...], preferred_element_type=jnp.float32) \
        + gcb_ref[...]
    out_ref[...] = jnp.maximum(z, 0.0)


@jax.jit
def kernel(x, bn_W, bn_b, gc_W, gc_b):
    n, in_f = x.shape
    b = n // _E
    bn_f = bn_W.shape[1]
    out_f = gc_W.shape[1]

    out = pl.pallas_call(
        functools.partial(_attn_kernel, _TOP_K),
        grid=(b,),
        in_specs=[
            pl.BlockSpec((_E, in_f), lambda i: (i, 0)),
            pl.BlockSpec((in_f, bn_f), lambda i: (0, 0)),
            pl.BlockSpec((1, bn_f), lambda i: (0, 0)),
            pl.BlockSpec((in_f, out_f), lambda i: (0, 0)),
            pl.BlockSpec((1, out_f), lambda i: (0, 0)),
        ],
        out_specs=pl.BlockSpec((_E, out_f), lambda i: (i, 0)),
        out_shape=jax.ShapeDtypeStruct((n, out_f), jnp.float32),
        compiler_params=pltpu.CompilerParams(
            dimension_semantics=("arbitrary",),
        ),
    )(x, bn_W, bn_b.reshape(1, bn_f), gc_W, gc_b.reshape(1, out_f))

    return out.reshape(b, _E, out_f)


# speedup vs baseline: 23.0315x; 1.0363x over previous
import functools

import jax
import jax.numpy as jnp
import numpy as np
from jax.experimental import pallas as pl
from jax.experimental.pallas import tpu as pltpu

_E = 1024
_TOP_K = 64
_INT32_MIN = np.int32(-(2 ** 31))
_INT32_MAXP = np.int32(2 ** 31 - 1)


def _search16_cols(key16, top_k):
    e_dim = key16.shape[0]

    def body(i, prefix):
        bitmask = jnp.left_shift(jnp.int32(1), jnp.int32(15) - i)
        trial = prefix | bitmask
        strial = (trial - 32768).astype(jnp.int16)
        cond = (key16 >= strial).astype(jnp.int16)
        c = cond[: e_dim // 2] + cond[e_dim // 2:]
        c = c[: e_dim // 4] + c[e_dim // 4:]
        c = c[: e_dim // 8] + c[e_dim // 8:]
        c = c[: e_dim // 16] + c[e_dim // 16:]
        cnt = jnp.sum(c.astype(jnp.int32), axis=0, keepdims=True)
        return jnp.where(cnt >= top_k, trial, prefix)

    prefix0 = jnp.zeros((1, key16.shape[1]), jnp.int32)
    prefix = jax.lax.fori_loop(0, 16, body, prefix0)
    return (prefix - 32768).astype(jnp.int16)


def _attn_kernel(top_k, x_ref, bnw_ref, bnb_ref, gcw_ref, gcb_ref, out_ref):
    h = x_ref[...]
    e_dim = h.shape[0]

    g = jnp.tanh(
        jnp.dot(h, bnw_ref[...], preferred_element_type=jnp.float32)
        + bnb_ref[...]
    )

    s = jax.lax.dot_general(
        g, g, (((1,), (1,)), ((), ())), preferred_element_type=jnp.float32
    )

    ex = jnp.exp(s - jnp.float32(g.shape[1]))
    half = e_dim // 2
    exc = ex[:half] + ex[half:]
    exc = exc[: half // 2] + exc[half // 2:]
    rsum = jnp.sum(exc, axis=0, keepdims=True)
    rdenom = jnp.transpose(1.0 / rsum)

    bits = jax.lax.bitcast_convert_type(s, jnp.int32)
    skey = bits ^ (jax.lax.shift_right_arithmetic(bits, 31) & _INT32_MAXP)

    hi = jax.lax.shift_right_arithmetic(skey, 16).astype(jnp.int16)
    lo = (skey ^ 0x8000).astype(jnp.int16)

    th_hi = _search16_cols(hi, top_k)

    lo_m_cols = jnp.where(
        hi > th_hi, jnp.int16(32767),
        jnp.where(hi < th_hi, jnp.int16(-32768), lo))
    th_lo = _search16_cols(lo_m_cols, top_k)

    th_hi_c = jnp.transpose(th_hi.astype(jnp.int32)).astype(jnp.int16)
    th_lo_c = jnp.transpose(th_lo.astype(jnp.int32)).astype(jnp.int16)
    lo_m_rows = jnp.where(
        hi > th_hi_c, jnp.int16(32767),
        jnp.where(hi < th_hi_c, jnp.int16(-32768), lo))
    keep = lo_m_rows >= th_lo_c

    row_ids = jax.lax.broadcasted_iota(jnp.int32, (e_dim, e_dim), 0)
    col_ids = jax.lax.broadcasted_iota(jnp.int32, (e_dim, e_dim), 1)
    keep = jnp.logical_and(keep, row_ids != col_ids)

    am = jnp.where(keep, ex, 0.0).astype(jnp.bfloat16)

    y = jnp.dot(am, h.astype(jnp.bfloat16),
                preferred_element_type=jnp.float32) * rdenom + h
    z = jnp.dot(y, gcw_ref[...], preferred_element_type=jnp.float32) \
        + gcb_ref[...]
    out_ref[...] = jnp.maximum(z, 0.0)


@jax.jit
def kernel(x, bn_W, bn_b, gc_W, gc_b):
    n, in_f = x.shape
    b = n // _E
    bn_f = bn_W.shape[1]
    out_f = gc_W.shape[1]

    out = pl.pallas_call(
        functools.partial(_attn_kernel, _TOP_K),
        grid=(b,),
        in_specs=[
            pl.BlockSpec((_E, in_f), lambda i: (i, 0)),
            pl.BlockSpec((in_f, bn_f), lambda i: (0, 0)),
            pl.BlockSpec((1, bn_f), lambda i: (0, 0)),
            pl.BlockSpec((in_f, out_f), lambda i: (0, 0)),
            pl.BlockSpec((1, out_f), lambda i: (0, 0)),
        ],
        out_specs=pl.BlockSpec((_E, out_f), lambda i: (i, 0)),
        out_shape=jax.ShapeDtypeStruct((n, out_f), jnp.float32),
        compiler_params=pltpu.CompilerParams(
            dimension_semantics=("arbitrary",),
        ),
    )(x, bn_W, bn_b.reshape(1, bn_f), gc_W, gc_b.reshape(1, out_f))

    return out.reshape(b, _E, out_f)


# speedup vs baseline: 23.6048x; 1.0249x over previous
import functools

import jax
import jax.numpy as jnp
import numpy as np
from jax.experimental import pallas as pl
from jax.experimental.pallas import tpu as pltpu

_E = 1024
_TOP_K = 64
_INT32_MIN = np.int32(-(2 ** 31))
_INT32_MAXP = np.int32(2 ** 31 - 1)


def _search16_cols(key16, top_k):
    e_dim = key16.shape[0]

    def body(i, prefix):
        bitmask = jnp.left_shift(jnp.int32(1), jnp.int32(15) - i)
        trial = prefix | bitmask
        strial = (trial - 32768).astype(jnp.int16)
        cond = (key16 >= strial).astype(jnp.int16)
        c = cond[: e_dim // 2] + cond[e_dim // 2:]
        c = c[: e_dim // 4] + c[e_dim // 4:]
        c = c[: e_dim // 8] + c[e_dim // 8:]
        c = c[: e_dim // 16] + c[e_dim // 16:]
        c = c[: e_dim // 32] + c[e_dim // 32:]
        c = c[: e_dim // 64] + c[e_dim // 64:]
        cnt = jnp.sum(c.astype(jnp.int32), axis=0, keepdims=True)
        return jnp.where(cnt >= top_k, trial, prefix)

    prefix0 = jnp.zeros((1, key16.shape[1]), jnp.int32)
    prefix = jax.lax.fori_loop(0, 16, body, prefix0)
    return (prefix - 32768).astype(jnp.int16)


def _attn_kernel(top_k, x_ref, bnw_ref, bnb_ref, gcw_ref, gcb_ref, nd_ref,
                 out_ref):
    h = x_ref[...]
    e_dim = h.shape[0]

    g = jnp.tanh(
        jnp.dot(h, bnw_ref[...], preferred_element_type=jnp.float32)
        + bnb_ref[...]
    )

    s = jax.lax.dot_general(
        g, g, (((1,), (1,)), ((), ())), preferred_element_type=jnp.float32
    )

    ex = jnp.exp(s - jnp.float32(g.shape[1]))
    half = e_dim // 2
    exc = ex[:half] + ex[half:]
    exc = exc[: half // 2] + exc[half // 2:]
    rsum = jnp.sum(exc, axis=0, keepdims=True)
    rdenom = jnp.transpose(1.0 / rsum)

    bits = jax.lax.bitcast_convert_type(s, jnp.int32)
    skey = bits ^ (jax.lax.shift_right_arithmetic(bits, 31) & _INT32_MAXP)

    hi = jax.lax.shift_right_arithmetic(skey, 16).astype(jnp.int16)
    lo = (skey ^ 0x8000).astype(jnp.int16)

    th_hi = _search16_cols(hi, top_k)

    lo_m_cols = jnp.where(
        hi > th_hi, jnp.int16(32767),
        jnp.where(hi < th_hi, jnp.int16(-32768), lo))
    th_lo = _search16_cols(lo_m_cols, top_k)

    th_hi_c = jnp.transpose(th_hi.astype(jnp.int32)).astype(jnp.int16)
    th_lo_c = jnp.transpose(th_lo.astype(jnp.int32)).astype(jnp.int16)
    lo_m_rows = jnp.where(
        hi > th_hi_c, jnp.int16(32767),
        jnp.where(hi < th_hi_c, jnp.int16(-32768), lo))
    keep = jnp.logical_and(lo_m_rows >= th_lo_c, nd_ref[...] != 0)

    am = jnp.where(keep, ex.astype(jnp.bfloat16), jnp.bfloat16(0.0))

    y = jnp.dot(am, h.astype(jnp.bfloat16),
                preferred_element_type=jnp.float32) * rdenom + h
    z = jnp.dot(y, gcw_ref[...], preferred_element_type=jnp.float32) \
        + gcb_ref[...]
    out_ref[...] = jnp.maximum(z, 0.0)


@jax.jit
def kernel(x, bn_W, bn_b, gc_W, gc_b):
    n, in_f = x.shape
    b = n // _E
    bn_f = bn_W.shape[1]
    out_f = gc_W.shape[1]

    notdiag = 1 - jnp.eye(_E, dtype=jnp.int16)

    out = pl.pallas_call(
        functools.partial(_attn_kernel, _TOP_K),
        grid=(b,),
        in_specs=[
            pl.BlockSpec((_E, in_f), lambda i: (i, 0)),
            pl.BlockSpec((in_f, bn_f), lambda i: (0, 0)),
            pl.BlockSpec((1, bn_f), lambda i: (0, 0)),
            pl.BlockSpec((in_f, out_f), lambda i: (0, 0)),
            pl.BlockSpec((1, out_f), lambda i: (0, 0)),
            pl.BlockSpec((_E, _E), lambda i: (0, 0)),
        ],
        out_specs=pl.BlockSpec((_E, out_f), lambda i: (i, 0)),
        out_shape=jax.ShapeDtypeStruct((n, out_f), jnp.float32),
        compiler_params=pltpu.CompilerParams(
            dimension_semantics=("arbitrary",),
        ),
    )(x, bn_W, bn_b.reshape(1, bn_f), gc_W, gc_b.reshape(1, out_f), notdiag)

    return out.reshape(b, _E, out_f)
